# baseline (device time: 165438 ns/iter reference)
import jax
import jax.numpy as jnp
from jax import lax
from jax.experimental import pallas as pl
from jax.experimental.pallas import tpu as pltpu


def kernel(x, A, B, C):
    Bb, S, D = x.shape
    N = A.shape[1]

    def body(x_ref, a_ref, b_ref, c_ref, out_ref, h_ref, send_sem, recv_sem):
        my_x = lax.axis_index("x")
        my_y = lax.axis_index("y")
        other_x = 1 - my_x

        barrier_sem = pltpu.get_barrier_semaphore()
        pl.semaphore_signal(
            barrier_sem, inc=1,
            device_id=(other_x, my_y),
            device_id_type=pl.DeviceIdType.MESH,
        )
        pl.semaphore_wait(barrier_sem, 1)

        dAT = jnp.exp(a_ref[:, :]).T.reshape(1, N, D)

        @pl.when(my_x == 0)
        def _():
            h_ref[...] = jnp.zeros((Bb, N, D), jnp.float32)

        @pl.when(my_x == 1)
        def _():
            recv = pltpu.make_async_remote_copy(
                src_ref=h_ref, dst_ref=h_ref,
                send_sem=send_sem, recv_sem=recv_sem,
                device_id=(0, my_y), device_id_type=pl.DeviceIdType.MESH,
            )
            recv.wait_recv()

        def step(t, h):
            x_t = x_ref[:, t, :]
            b_t = b_ref[:, t, :]
            c_t = c_ref[:, t, :]
            h = h * dAT + x_t[:, None, :] * b_t[:, :, None]
            out_ref[:, t, :] = jnp.sum(h * c_t[:, :, None], axis=1)
            return h

        h_final = lax.fori_loop(0, S, step, h_ref[...])

        @pl.when(my_x == 0)
        def _():
            h_ref[...] = h_final
            send = pltpu.make_async_remote_copy(
                src_ref=h_ref, dst_ref=h_ref,
                send_sem=send_sem, recv_sem=recv_sem,
                device_id=(1, my_y), device_id_type=pl.DeviceIdType.MESH,
            )
            send.start()
            send.wait_send()

    return pl.pallas_call(
        body,
        out_shape=jax.ShapeDtypeStruct((Bb, S, D), jnp.float32),
        in_specs=[pl.BlockSpec(memory_space=pltpu.VMEM)] * 4,
        out_specs=pl.BlockSpec(memory_space=pltpu.VMEM),
        scratch_shapes=[
            pltpu.VMEM((Bb, N, D), jnp.float32),
            pltpu.SemaphoreType.DMA,
            pltpu.SemaphoreType.DMA,
        ],
        compiler_params=pltpu.CompilerParams(collective_id=0),
    )(x, A, B, C)
